# baseline (speedup 1.0000x reference)
import functools

import jax
import jax.numpy as jnp
from jax import lax
from jax.experimental import pallas as pl
from jax.experimental.pallas import tpu as pltpu
from jax.experimental.pallas import tpu_sc as plsc

D_EMBED = 64
EMB_SCALE = 8.0
NUM_WORKERS = 32
BT = 128
HIST = 200
NBUF = 4
LANES = 16


def _sc_embed(idx4, emb_table):
    mesh = plsc.VectorSubcoreMesh(core_axis_name="c", subcore_axis_name="s")

    row_buf = pltpu.VMEM((BT, D_EMBED), jnp.float32)
    t_buf = pltpu.VMEM((D_EMBED * BT,), jnp.float32)
    scratch = (
        [pltpu.VMEM((HIST // 8, 8, BT), jnp.int32)]
        + [row_buf] * NBUF
        + [t_buf] * NBUF
        + [pltpu.SemaphoreType.DMA] * (2 * NBUF)
    )

    @functools.partial(
        pl.kernel,
        mesh=mesh,
        out_type=jax.ShapeDtypeStruct(
            (HIST, 8, NUM_WORKERS, 8 * BT), jnp.float32
        ),
        scratch_types=scratch,
        compiler_params=pltpu.CompilerParams(
            use_tc_tiling_on_sc=False, needs_layout_passes=False
        ),
    )
    def k(idx_hbm, table_hbm, out_hbm, idx_v, *bufs):
        rbufs = bufs[:NBUF]
        tbufs = bufs[NBUF : 2 * NBUF]
        gsems = bufs[2 * NBUF : 3 * NBUF]
        ssems = bufs[3 * NBUF :]

        wid = lax.axis_index("s") * 2 + lax.axis_index("c")
        pltpu.sync_copy(idx_hbm.at[:, wid], idx_v)

        iota = lax.iota(jnp.int32, LANES)
        dflat = [(iota + 16 * q) * BT for q in range(D_EMBED // LANES)]

        def gather(h, b):
            pltpu.async_copy(
                table_hbm.at[idx_v.at[h >> 3, h & 7]], rbufs[b], gsems[b]
            )

        def gather_wait(h, b):
            pltpu.make_async_copy(
                table_hbm.at[idx_v.at[h >> 3, h & 7]], rbufs[b], gsems[b]
            ).wait()

        def store(h, b):
            for dt in range(8):
                pltpu.async_copy(
                    tbufs[b].at[pl.ds(dt * 8 * BT, 8 * BT)],
                    out_hbm.at[h, dt, wid],
                    ssems[b],
                )

        def store_wait(h, b):
            for dt in range(8):
                pltpu.make_async_copy(
                    tbufs[b].at[pl.ds(dt * 8 * BT, 8 * BT)],
                    out_hbm.at[h, dt, wid],
                    ssems[b],
                ).wait()

        def transpose_scale(b):
            r, t = rbufs[b], tbufs[b]

            def body(i, carry):
                for q in range(D_EMBED // LANES):
                    vals = r[i, pl.ds(16 * q, LANES)] * EMB_SCALE
                    plsc.store_scatter(t, [dflat[q] + i], vals)
                return carry

            lax.fori_loop(0, BT, body, 0)

        for b in range(NBUF):
            gather(b, b)

        rounds = HIST // NBUF

        def visit(rnd, carry):
            for b in range(NBUF):
                h = rnd * NBUF + b
                gather_wait(h, b)

                @pl.when(rnd > 0)
                def _():
                    store_wait(h - NBUF, b)

                transpose_scale(b)
                store(h, b)

                @pl.when(rnd < rounds - 1)
                def _():
                    gather(h + NBUF, b)

            return carry

        lax.fori_loop(0, rounds, visit, 0)

        for b in range(NBUF):
            store_wait(HIST - NBUF + b, b)

    return k(idx4, emb_table)


def kernel(inp, emb_table):
    batch, hist = inp.shape
    idx4 = inp.reshape(NUM_WORKERS, BT, hist // 8, 8).transpose(2, 0, 3, 1)
    out4 = _sc_embed(idx4, emb_table)
    out5 = out4.reshape(hist, 8, NUM_WORKERS, 8, BT)
    out = out5.transpose(2, 4, 0, 1, 3).reshape(batch, hist, D_EMBED)
    return out


# speedup vs baseline: 1.5378x; 1.5378x over previous
import functools

import jax
import jax.numpy as jnp
from jax import lax
from jax.experimental import pallas as pl
from jax.experimental.pallas import tpu as pltpu
from jax.experimental.pallas import tpu_sc as plsc

D_EMBED = 64
EMB_SCALE = 8.0
NUM_WORKERS = 32
BT = 128
HIST = 200
NBUF = 4
LANES = 16


def _sc_embed(idx4, emb_table):
    mesh = plsc.VectorSubcoreMesh(core_axis_name="c", subcore_axis_name="s")

    row_buf = pltpu.VMEM((BT, D_EMBED), jnp.float32)
    t_buf = pltpu.VMEM((D_EMBED, BT + 1), jnp.float32)
    scratch = (
        [pltpu.VMEM((HIST // 8, 8, BT), jnp.int32)]
        + [row_buf] * NBUF
        + [t_buf] * NBUF
        + [pltpu.SemaphoreType.DMA] * (2 * NBUF)
    )

    @functools.partial(
        pl.kernel,
        mesh=mesh,
        out_type=jax.ShapeDtypeStruct(
            (HIST, 8, NUM_WORKERS, 8, BT), jnp.float32
        ),
        scratch_types=scratch,
        compiler_params=pltpu.CompilerParams(
            use_tc_tiling_on_sc=False, needs_layout_passes=False
        ),
    )
    def k(idx_hbm, table_hbm, out_hbm, idx_v, *bufs):
        rbufs = bufs[:NBUF]
        tbufs = bufs[NBUF : 2 * NBUF]
        gsems = bufs[2 * NBUF : 3 * NBUF]
        ssems = bufs[3 * NBUF :]

        wid = lax.axis_index("s") * 2 + lax.axis_index("c")
        pltpu.sync_copy(idx_hbm.at[:, wid], idx_v)

        iota = lax.iota(jnp.int32, LANES)
        dvecs = [iota + 16 * q for q in range(D_EMBED // LANES)]

        def gather(h, b):
            pltpu.async_copy(
                table_hbm.at[idx_v.at[h >> 3, h & 7]], rbufs[b], gsems[b]
            )

        def gather_wait(h, b):
            pltpu.make_async_copy(
                table_hbm.at[idx_v.at[h >> 3, h & 7]], rbufs[b], gsems[b]
            ).wait()

        def store(h, b):
            for dt in range(8):
                pltpu.async_copy(
                    tbufs[b].at[pl.ds(dt * 8, 8), pl.ds(0, BT)],
                    out_hbm.at[h, dt, wid],
                    ssems[b],
                )

        def store_wait(h, b):
            for dt in range(8):
                pltpu.make_async_copy(
                    tbufs[b].at[pl.ds(dt * 8, 8), pl.ds(0, BT)],
                    out_hbm.at[h, dt, wid],
                    ssems[b],
                ).wait()

        def transpose_scale(b):
            r, t = rbufs[b], tbufs[b]

            def body(i, carry):
                bvec = jnp.full((LANES,), 0, jnp.int32) + i
                for q in range(D_EMBED // LANES):
                    vals = r[i, pl.ds(16 * q, LANES)] * EMB_SCALE
                    plsc.store_scatter(t, [dvecs[q], bvec], vals)
                return carry

            lax.fori_loop(0, BT, body, 0)

        for b in range(NBUF):
            gather(b, b)

        rounds = HIST // NBUF

        def visit(rnd, carry):
            for b in range(NBUF):
                h = rnd * NBUF + b
                gather_wait(h, b)

                @pl.when(rnd > 0)
                def _():
                    store_wait(h - NBUF, b)

                transpose_scale(b)
                store(h, b)

                @pl.when(rnd < rounds - 1)
                def _():
                    gather(h + NBUF, b)

            return carry

        lax.fori_loop(0, rounds, visit, 0)

        for b in range(NBUF):
            store_wait(HIST - NBUF + b, b)

    return k(idx4, emb_table)


def kernel(inp, emb_table):
    batch, hist = inp.shape
    idx4 = inp.reshape(NUM_WORKERS, BT, hist // 8, 8).transpose(2, 0, 3, 1)
    out5 = _sc_embed(idx4, emb_table)
    out = out5.transpose(2, 4, 0, 1, 3).reshape(batch, hist, D_EMBED)
    return out


# speedup vs baseline: 2.3266x; 1.5130x over previous
import functools

import jax
import jax.numpy as jnp
from jax import lax
from jax.experimental import pallas as pl
from jax.experimental.pallas import tpu as pltpu
from jax.experimental.pallas import tpu_sc as plsc

D_EMBED = 64
EMB_SCALE = 8.0
NUM_WORKERS = 32
BT = 128
HIST = 200
NBUF = 4
LANES = 16


def _sc_embed(idx4, emb_table):
    mesh = plsc.VectorSubcoreMesh(core_axis_name="c", subcore_axis_name="s")

    row_buf = pltpu.VMEM((BT, D_EMBED), jnp.float32)
    t_buf = pltpu.VMEM((D_EMBED, BT + 1), jnp.float32)
    scratch = (
        [pltpu.VMEM((HIST // 8, 8, BT), jnp.int32)]
        + [row_buf] * NBUF
        + [t_buf] * NBUF
        + [pltpu.SemaphoreType.DMA] * (2 * NBUF)
    )

    @functools.partial(
        pl.kernel,
        mesh=mesh,
        out_type=jax.ShapeDtypeStruct(
            (HIST, 8, NUM_WORKERS, 8, BT), jnp.float32
        ),
        scratch_types=scratch,
        compiler_params=pltpu.CompilerParams(
            use_tc_tiling_on_sc=False, needs_layout_passes=False
        ),
    )
    def k(idx_hbm, table_hbm, out_hbm, idx_v, *bufs):
        rbufs = bufs[:NBUF]
        tbufs = bufs[NBUF : 2 * NBUF]
        gsems = bufs[2 * NBUF : 3 * NBUF]
        ssems = bufs[3 * NBUF :]

        wid = lax.axis_index("s") * 2 + lax.axis_index("c")
        pltpu.sync_copy(idx_hbm.at[:, wid], idx_v)

        iota = lax.iota(jnp.int32, LANES)
        dvecs = [iota + 16 * q for q in range(D_EMBED // LANES)]

        def gather(h, b):
            pltpu.async_copy(
                table_hbm.at[idx_v.at[h >> 3, h & 7]], rbufs[b], gsems[b]
            )

        def gather_wait(h, b):
            pltpu.make_async_copy(
                table_hbm.at[idx_v.at[h >> 3, h & 7]], rbufs[b], gsems[b]
            ).wait()

        def store(h, b):
            for dt in range(8):
                pltpu.async_copy(
                    tbufs[b].at[pl.ds(dt * 8, 8), pl.ds(0, BT)],
                    out_hbm.at[h, dt, wid],
                    ssems[b],
                )

        def store_wait(h, b):
            for dt in range(8):
                pltpu.make_async_copy(
                    tbufs[b].at[pl.ds(dt * 8, 8), pl.ds(0, BT)],
                    out_hbm.at[h, dt, wid],
                    ssems[b],
                ).wait()

        def transpose_scale(b):
            r, t = rbufs[b], tbufs[b]

            @plsc.parallel_loop(0, BT, 1, unroll=8)
            def body(i):
                bvec = jnp.full((LANES,), 0, jnp.int32) + i
                for q in range(D_EMBED // LANES):
                    vals = r[i, pl.ds(16 * q, LANES)] * EMB_SCALE
                    plsc.store_scatter(t, [dvecs[q], bvec], vals)

        for b in range(NBUF):
            gather(b, b)

        rounds = HIST // NBUF

        def visit(rnd, carry):
            for b in range(NBUF):
                h = rnd * NBUF + b
                gather_wait(h, b)

                @pl.when(rnd > 0)
                def _():
                    store_wait(h - NBUF, b)

                transpose_scale(b)
                store(h, b)

                @pl.when(rnd < rounds - 1)
                def _():
                    gather(h + NBUF, b)

            return carry

        lax.fori_loop(0, rounds, visit, 0)

        for b in range(NBUF):
            store_wait(HIST - NBUF + b, b)

    return k(idx4, emb_table)


def kernel(inp, emb_table):
    batch, hist = inp.shape
    idx4 = inp.reshape(NUM_WORKERS, BT, hist // 8, 8).transpose(2, 0, 3, 1)
    out5 = _sc_embed(idx4, emb_table)
    out = out5.transpose(2, 4, 0, 1, 3).reshape(batch, hist, D_EMBED)
    return out


# speedup vs baseline: 2.3288x; 1.0010x over previous
import functools

import jax
import jax.numpy as jnp
from jax import lax
from jax.experimental import pallas as pl
from jax.experimental.pallas import tpu as pltpu
from jax.experimental.pallas import tpu_sc as plsc

D_EMBED = 64
EMB_SCALE = 8.0
NUM_WORKERS = 32
BT = 128
HIST = 200
NBUF = 4
LANES = 16


def _sc_embed(idx4, emb_table):
    mesh = plsc.VectorSubcoreMesh(core_axis_name="c", subcore_axis_name="s")

    row_buf = pltpu.VMEM((BT, D_EMBED), jnp.float32)
    t_buf = pltpu.VMEM((D_EMBED, BT + 1), jnp.float32)
    scratch = (
        [pltpu.VMEM((HIST // 8, 8, BT), jnp.int32)]
        + [row_buf] * NBUF
        + [t_buf] * NBUF
        + [pltpu.SemaphoreType.DMA] * (2 * NBUF)
    )

    @functools.partial(
        pl.kernel,
        mesh=mesh,
        out_type=jax.ShapeDtypeStruct(
            (HIST, 8, NUM_WORKERS, 8, BT), jnp.float32
        ),
        scratch_types=scratch,
        compiler_params=pltpu.CompilerParams(
            use_tc_tiling_on_sc=False,
            needs_layout_passes=False,
            skip_device_barrier=True,
        ),
    )
    def k(idx_hbm, table_hbm, out_hbm, idx_v, *bufs):
        rbufs = bufs[:NBUF]
        tbufs = bufs[NBUF : 2 * NBUF]
        gsems = bufs[2 * NBUF : 3 * NBUF]
        ssems = bufs[3 * NBUF :]

        wid = lax.axis_index("s") * 2 + lax.axis_index("c")
        pltpu.sync_copy(idx_hbm.at[:, wid], idx_v)

        iota = lax.iota(jnp.int32, LANES)
        dvecs = [iota + 16 * q for q in range(D_EMBED // LANES)]

        def gather(h, b):
            pltpu.async_copy(
                table_hbm.at[idx_v.at[h >> 3, h & 7]], rbufs[b], gsems[b]
            )

        def gather_wait(h, b):
            pltpu.make_async_copy(
                table_hbm.at[idx_v.at[h >> 3, h & 7]], rbufs[b], gsems[b]
            ).wait()

        def store(h, b):
            for dt in range(8):
                pltpu.async_copy(
                    tbufs[b].at[pl.ds(dt * 8, 8), pl.ds(0, BT)],
                    out_hbm.at[h, dt, wid],
                    ssems[b],
                )

        def store_wait(h, b):
            for dt in range(8):
                pltpu.make_async_copy(
                    tbufs[b].at[pl.ds(dt * 8, 8), pl.ds(0, BT)],
                    out_hbm.at[h, dt, wid],
                    ssems[b],
                ).wait()

        def transpose_scale(b):
            r, t = rbufs[b], tbufs[b]

            @plsc.parallel_loop(0, BT, 1, unroll=8)
            def body(i):
                bvec = jnp.full((LANES,), 0, jnp.int32) + i
                for q in range(D_EMBED // LANES):
                    vals = r[i, pl.ds(16 * q, LANES)] * EMB_SCALE
                    plsc.store_scatter(t, [dvecs[q], bvec], vals)

        for b in range(NBUF):
            gather(b, b)

        rounds = HIST // NBUF

        def visit(rnd, carry):
            for b in range(NBUF):
                h = rnd * NBUF + b
                gather_wait(h, b)

                @pl.when(rnd > 0)
                def _():
                    store_wait(h - NBUF, b)

                transpose_scale(b)
                store(h, b)

                @pl.when(rnd < rounds - 1)
                def _():
                    gather(h + NBUF, b)

            return carry

        lax.fori_loop(0, rounds, visit, 0)

        for b in range(NBUF):
            store_wait(HIST - NBUF + b, b)

    return k(idx4, emb_table)


def kernel(inp, emb_table):
    batch, hist = inp.shape
    idx4 = inp.reshape(NUM_WORKERS, BT, hist // 8, 8).transpose(2, 0, 3, 1)
    out5 = _sc_embed(idx4, emb_table)
    out = out5.transpose(2, 4, 0, 1, 3).reshape(batch, hist, D_EMBED)
    return out
